# baseline (speedup 1.0000x reference)
import functools
import math

import jax
import jax.numpy as jnp
from jax import lax
from jax.experimental import pallas as pl
from jax.experimental.pallas import tpu as pltpu
from jax.experimental.pallas import tpu_sc as plsc

_N = 10000
_E = 160000
_D = 256
_H = 128
_CHUNK = 128
_TILES = 16
_RAW_CHUNKS = 80
_EPAD = _TILES * _RAW_CHUNKS * _CHUNK
_NB = 5
_R = 2048
_ACC_ROWS = 2056
_SEG = 2048
_CAP = 12288
_INV_SQRT = 1.0 / math.sqrt(1.0 + 1e-5)
_TRASH = _R


def _lanes():
  return lax.iota(jnp.int32, 16)


def _prefix_incl(m):
  p = m.astype(jnp.int32)
  lanes = _lanes()
  for sh in (1, 2, 4, 8):
    idx = jnp.maximum(lanes - sh, 0)
    shifted = p.at[idx].get(mode="promise_in_bounds")
    p = p + jnp.where(lanes >= sh, shifted, 0)
  return p


def _splat_last(p):
  idx = jnp.full((16,), 15, jnp.int32)
  return p.at[idx].get(mode="promise_in_bounds")


def _scalarize(vec_splat, nbits):
  k = jnp.int32(0)
  for bit in range(nbits):
    hasbit = jnp.any(jnp.bitwise_and(vec_splat, jnp.int32(1 << bit)) != 0)
    k = k + hasbit.astype(jnp.int32) * (1 << bit)
  return k


def _bucket_edges(src_r, dst_r):
  mesh = plsc.VectorSubcoreMesh(core_axis_name="c", subcore_axis_name="s")
  nlists = _TILES * _NB

  @functools.partial(
      pl.kernel,
      mesh=mesh,
      out_type=[
          jax.ShapeDtypeStruct((nlists * _CAP,), jnp.int32),
          jax.ShapeDtypeStruct((nlists * _CAP,), jnp.int32),
          jax.ShapeDtypeStruct((_TILES, 8, 16), jnp.int32),
      ],
      scratch_types=[
          pltpu.VMEM((8, _CHUNK), jnp.int32),
          pltpu.VMEM((8, _CHUNK), jnp.int32),
          pltpu.VMEM((8, _CHUNK), jnp.int32),
          pltpu.VMEM((8, _CHUNK), jnp.int32),
          *([pltpu.VMEM((_SEG + 32,), jnp.int32)] * (2 * _NB)),
          pltpu.VMEM((8, 16), jnp.int32),
          pltpu.SemaphoreType.DMA,
      ],
  )
  def bucket(src_hbm, dst_hbm, sbkt, dbkt, counts,
             sraw0, draw0, sraw1, draw1,
             sbb0, dbb0, sbb1, dbb1, sbb2, dbb2, sbb3, dbb3, sbb4, dbb4,
             cbuf, semi):
    sbbs = (sbb0, sbb1, sbb2, sbb3, sbb4)
    dbbs = (dbb0, dbb1, dbb2, dbb3, dbb4)
    c = lax.axis_index("c")
    s = lax.axis_index("s")
    sraws = (sraw0, sraw1)
    draws = (draw0, draw1)

    def issue_raw(blk, par):
      base = s * _RAW_CHUNKS + blk * 8
      pltpu.async_copy(src_hbm.at[pl.ds(base, 8)], sraws[par], semi)
      pltpu.async_copy(dst_hbm.at[pl.ds(base, 8)], draws[par], semi)

    def wait_raw(par):
      pltpu.make_async_copy(src_hbm.at[pl.ds(0, 8)], sraws[par], semi).wait()
      pltpu.make_async_copy(dst_hbm.at[pl.ds(0, 8)], draws[par], semi).wait()

    @pl.when(c == 0)
    def _():
      issue_raw(0, 0)
      zero_v = jnp.zeros((16,), jnp.int32)
      carry0 = (zero_v,) * _NB + (jnp.int32(0),) * _NB
      for blk in range(_RAW_CHUNKS // 8):
        par = blk % 2
        wait_raw(par)
        if blk + 1 < _RAW_CHUNKS // 8:
          issue_raw(blk + 1, 1 - par)

        def row_body(r, carry):
          def sub_body(v, carry):
            s16 = sraws[par][r, pl.ds(v * 16, 16)]
            d16 = draws[par][r, pl.ds(v * 16, 16)]
            b16 = lax.shift_right_logical(d16, 11)
            dl16 = lax.bitwise_and(d16, jnp.int32(_R - 1))
            nn = []
            ff = []
            for b in range(_NB):
              n_v = carry[b]
              f_b = carry[_NB + b]
              m = b16 == b
              prefix = _prefix_incl(m)
              idxv = jnp.where(m, n_v + prefix - 1, jnp.int32(_SEG + 16))
              plsc.store_scatter(sbbs[b], [idxv], s16)
              plsc.store_scatter(dbbs[b], [idxv], dl16)
              n_v = n_v + _splat_last(prefix)
              spill = jnp.any(n_v >= _SEG)

              @pl.when(spill)
              def _():
                off = (s * _NB + b) * _CAP + f_b * _SEG
                pltpu.sync_copy(sbbs[b].at[pl.ds(0, _SEG)],
                                sbkt.at[pl.ds(off, _SEG)])
                pltpu.sync_copy(dbbs[b].at[pl.ds(0, _SEG)],
                                dbkt.at[pl.ds(off, _SEG)])
                sv = sbbs[b][pl.ds(_SEG, 16)]
                dv = dbbs[b][pl.ds(_SEG, 16)]
                sbbs[b][pl.ds(0, 16)] = sv
                dbbs[b][pl.ds(0, 16)] = dv

              spill_i = spill.astype(jnp.int32)
              nn.append(n_v - spill_i * _SEG)
              ff.append(f_b + spill_i)
            return tuple(nn) + tuple(ff)
          return lax.fori_loop(0, 8, sub_body, carry)
        carry0 = lax.fori_loop(0, 8, row_body, carry0)

      lanes = _lanes()
      for b in range(_NB):
        n_v = carry0[b]
        f_b = carry0[_NB + b]
        np128v = lax.bitwise_and(n_v + 127, jnp.int32(~127))

        @pl.when(jnp.any(n_v > 0))
        def _():
          for w in range(8):
            iv = n_v + lanes + 16 * w
            iv = jnp.where(iv < np128v, iv, jnp.int32(_SEG + 16))
            plsc.store_scatter(sbbs[b], [iv], jnp.zeros((16,), jnp.int32))
            plsc.store_scatter(dbbs[b], [iv],
                               jnp.full((16,), _TRASH, jnp.int32))
          off = (s * _NB + b) * _CAP + f_b * _SEG
          pltpu.sync_copy(sbbs[b].at[pl.ds(0, _SEG)],
                          sbkt.at[pl.ds(off, _SEG)])
          pltpu.sync_copy(dbbs[b].at[pl.ds(0, _SEG)],
                          dbkt.at[pl.ds(off, _SEG)])

        k_v = (lax.shift_right_logical(np128v, 7)
               + f_b * (_SEG // _CHUNK))
        cbuf[b, pl.ds(0, 16)] = k_v
      for b in range(_NB, 8):
        cbuf[b, pl.ds(0, 16)] = jnp.zeros((16,), jnp.int32)
      pltpu.sync_copy(cbuf, counts.at[s])

  return bucket(src_r, dst_r)


def _seg_aggregate(x_lo, x_hi, sbkt, dbkt, counts):
  mesh = plsc.VectorSubcoreMesh(core_axis_name="c", subcore_axis_name="s")

  @functools.partial(
      pl.kernel,
      mesh=mesh,
      out_type=[jax.ShapeDtypeStruct((_NB * _R, _H), jnp.float32)] * 2,
      scratch_types=[
          pltpu.VMEM((_CHUNK, _H), jnp.float32),
          pltpu.VMEM((_CHUNK, _H), jnp.float32),
          pltpu.VMEM((1, _CHUNK), jnp.int32),
          pltpu.VMEM((1, _CHUNK), jnp.int32),
          pltpu.VMEM((1, _CHUNK), jnp.int32),
          pltpu.VMEM((1, _CHUNK), jnp.int32),
          pltpu.VMEM((8, 16), jnp.int32),
          pltpu.VMEM_SHARED((_N, _H), jnp.float32),
          pltpu.VMEM_SHARED((_ACC_ROWS, _H), jnp.float32),
          pltpu.SemaphoreType.DMA,
          pltpu.SemaphoreType.DMA,
          pltpu.SemaphoreType.DMA,
      ],
  )
  def agg(xlo_hbm, xhi_hbm, sbkt, dbkt, counts, out_lo, out_hi,
          rb0, rb1, sA, dA, sB, dB, cbuf, table, acc, sem0, sem1, semi):
    c = lax.axis_index("c")
    s = lax.axis_index("s")

    @pl.when(s < 15)
    def _():
      @pl.when(c == 0)
      def _():
        pltpu.sync_copy(xlo_hbm.at[pl.ds(s * 632, 632)],
                        table.at[pl.ds(s * 632, 632)])

      @pl.when(c == 1)
      def _():
        pltpu.sync_copy(xhi_hbm.at[pl.ds(s * 632, 632)],
                        table.at[pl.ds(s * 632, 632)])

    @pl.when(s == 15)
    def _():
      @pl.when(c == 0)
      def _():
        pltpu.sync_copy(xlo_hbm.at[pl.ds(9480, 520)],
                        table.at[pl.ds(9480, 520)])

      @pl.when(c == 1)
      def _():
        pltpu.sync_copy(xhi_hbm.at[pl.ds(9480, 520)],
                        table.at[pl.ds(9480, 520)])

    pltpu.sync_copy(counts.at[s], cbuf)
    kbs = []
    for p in range(_NB):
      kbs.append(cbuf[p, pl.ds(0, 16)][0])

    def wait_idx():
      pltpu.make_async_copy(sbkt.at[pl.ds(0, _CHUNK)], sA.at[0], semi).wait()
      pltpu.make_async_copy(sbkt.at[pl.ds(0, _CHUNK)], dA.at[0], semi).wait()

    def wait_rb(buf, sm):
      pltpu.make_async_copy(xlo_hbm.at[pl.ds(0, _CHUNK)], buf, sm).wait()

    out_refs = (out_lo, out_hi)
    for p in range(_NB):
      base = (s * _NB + p) * _CAP

      def issue_idx(j, sbuf, dbuf):
        pltpu.async_copy(sbkt.at[pl.ds(base + j * _CHUNK, _CHUNK)],
                         sbuf.at[0], semi)
        pltpu.async_copy(dbkt.at[pl.ds(base + j * _CHUNK, _CHUNK)],
                         dbuf.at[0], semi)

      def zbody(i, carry):
        for k in range(_H // 16):
          rb0[i, pl.ds(k * 16, 16)] = jnp.zeros((16,), jnp.float32)
        return carry
      lax.fori_loop(0, _CHUNK, zbody, 0)
      pltpu.sync_copy(rb0, acc.at[pl.ds(s * _CHUNK, _CHUNK)])

      @pl.when(s == 0)
      def _():
        pltpu.sync_copy(rb0.at[pl.ds(0, 8)], acc.at[pl.ds(_R, 8)])
      plsc.subcore_barrier()

      kb = kbs[p]
      issue_idx(0, sA, dA)
      wait_idx()

      @pl.when(kb > 0)
      def _():
        pltpu.async_copy(table.at[sA.at[0]], rb0, sem0)

      @pl.when(kb > 1)
      def _():
        issue_idx(1, sB, dB)

      def body(i, carry):
        j0 = 2 * i

        @pl.when(j0 + 1 < kb)
        def _():
          wait_idx()
        wait_rb(rb0, sem0)

        @pl.when(j0 + 1 < kb)
        def _():
          pltpu.async_copy(table.at[sB.at[0]], rb1, sem1)
        pltpu.sync_copy(rb0, acc.at[dA.at[0]], add=True)

        @pl.when(j0 + 2 < kb)
        def _():
          issue_idx(j0 + 2, sA, dA)

        @pl.when(j0 + 2 < kb)
        def _():
          wait_idx()

        @pl.when(j0 + 1 < kb)
        def _():
          wait_rb(rb1, sem1)

          @pl.when(j0 + 2 < kb)
          def _():
            pltpu.async_copy(table.at[sA.at[0]], rb0, sem0)
          pltpu.sync_copy(rb1, acc.at[dB.at[0]], add=True)

          @pl.when(j0 + 3 < kb)
          def _():
            issue_idx(j0 + 3, sB, dB)
        return carry
      lax.fori_loop(0, lax.shift_right_logical(kb + 1, 1), body, 0)

      plsc.subcore_barrier()
      for cc in range(2):
        @pl.when(c == cc)
        def _():
          pltpu.sync_copy(acc.at[pl.ds(s * _CHUNK, _CHUNK)],
                          out_refs[cc].at[pl.ds(p * _R + s * _CHUNK, _CHUNK)])
      plsc.subcore_barrier()

  return agg(x_lo, x_hi, sbkt, dbkt, counts)


_BN_ROWS = 1000


def _mm_bn_relu(a_lo, a_hi, w, g, b):
  def body(lo_ref, hi_ref, w_ref, g_ref, b_ref, olo_ref, ohi_ref):
    a = jnp.concatenate([lo_ref[...], hi_ref[...]], axis=1)
    z = jnp.dot(a, w_ref[...], preferred_element_type=jnp.float32)
    h = jnp.maximum(z * (g_ref[...] * _INV_SQRT) + b_ref[...], 0.0)
    olo_ref[...] = h[:, :_H]
    ohi_ref[...] = h[:, _H:]

  return pl.pallas_call(
      body,
      grid=(_N // _BN_ROWS,),
      in_specs=[
          pl.BlockSpec((_BN_ROWS, _H), lambda i: (i, 0)),
          pl.BlockSpec((_BN_ROWS, _H), lambda i: (i, 0)),
          pl.BlockSpec((_D, _D), lambda i: (0, 0)),
          pl.BlockSpec((1, _D), lambda i: (0, 0)),
          pl.BlockSpec((1, _D), lambda i: (0, 0)),
      ],
      out_specs=[pl.BlockSpec((_BN_ROWS, _H), lambda i: (i, 0))] * 2,
      out_shape=[jax.ShapeDtypeStruct((_N, _H), jnp.float32)] * 2,
  )(a_lo, a_hi, w, g, b)


def _mm_logsoftmax(a_lo, a_hi, w):
  def body(lo_ref, hi_ref, w_ref, o_ref):
    a = jnp.concatenate([lo_ref[...], hi_ref[...]], axis=1)
    z = jnp.dot(a, w_ref[...], preferred_element_type=jnp.float32)
    m = jnp.max(z, axis=1, keepdims=True)
    lse = jnp.log(jnp.sum(jnp.exp(z - m), axis=1, keepdims=True)) + m
    o_ref[...] = z - lse

  return pl.pallas_call(
      body,
      grid=(_N // _BN_ROWS,),
      in_specs=[
          pl.BlockSpec((_BN_ROWS, _H), lambda i: (i, 0)),
          pl.BlockSpec((_BN_ROWS, _H), lambda i: (i, 0)),
          pl.BlockSpec((_D, _D), lambda i: (0, 0)),
      ],
      out_specs=pl.BlockSpec((_BN_ROWS, _D), lambda i: (i, 0)),
      out_shape=jax.ShapeDtypeStruct((_N, _D), jnp.float32),
  )(a_lo, a_hi, w)


def _bucket_edges_xla_DEBUG(src_r, dst_r):
  src = src_r.reshape(-1)
  dst = dst_r.reshape(-1)
  n = src.shape[0]
  tile = jnp.arange(n, dtype=jnp.int32) // (n // _TILES)
  b = jnp.right_shift(dst, 11)
  key = tile * _NB + b
  order = jnp.argsort(key, stable=True)
  key_s = key[order]
  counts_k = jnp.bincount(key, length=_TILES * _NB).astype(jnp.int32)
  starts = jnp.cumsum(counts_k) - counts_k
  within = jnp.arange(n, dtype=jnp.int32) - starts[key_s]
  slot = key_s * _CAP + within
  sbkt = jnp.zeros((_TILES * _NB * _CAP,), jnp.int32).at[slot].set(src[order])
  dbkt = jnp.full((_TILES * _NB * _CAP,), _TRASH, jnp.int32).at[slot].set(
      jnp.bitwise_and(dst, _R - 1)[order])
  kchunks = (counts_k + _CHUNK - 1) // _CHUNK
  counts = jnp.zeros((_TILES, 8, 16), jnp.int32).at[:, :_NB, :].set(
      kchunks.reshape(_TILES, _NB)[:, :, None])
  return sbkt, dbkt, counts


def kernel(x, edge_index, W0, W1, W2, gamma0, beta0, gamma1, beta1):
  x = x.astype(jnp.float32)
  src = edge_index[0].astype(jnp.int32)
  dst = edge_index[1].astype(jnp.int32)
  pad = _EPAD - _E
  src_r = jnp.concatenate([src, jnp.zeros((pad,), jnp.int32)]).reshape(-1, _CHUNK)
  dst_r = jnp.concatenate([dst, jnp.full((pad,), _N, jnp.int32)]).reshape(-1, _CHUNK)

  x_lo, x_hi = x[:, :_H], x[:, _H:]
  g0, b0 = gamma0.reshape(1, _D), beta0.reshape(1, _D)
  g1, b1 = gamma1.reshape(1, _D), beta1.reshape(1, _D)

  sbkt, dbkt, counts = _bucket_edges_xla_DEBUG(src_r, dst_r)
  a_lo, a_hi = _seg_aggregate(x_lo, x_hi, sbkt, dbkt, counts)
  h_lo, h_hi = _mm_bn_relu(a_lo, a_hi, W0, g0, b0)
  a_lo, a_hi = _seg_aggregate(h_lo, h_hi, sbkt, dbkt, counts)
  h_lo, h_hi = _mm_bn_relu(a_lo, a_hi, W1, g1, b1)
  a_lo, a_hi = _seg_aggregate(h_lo, h_hi, sbkt, dbkt, counts)
  return _mm_logsoftmax(a_lo, a_hi, W2)


# speedup vs baseline: 3.7091x; 3.7091x over previous
import functools
import math

import jax
import jax.numpy as jnp
from jax import lax
from jax.experimental import pallas as pl
from jax.experimental.pallas import tpu as pltpu
from jax.experimental.pallas import tpu_sc as plsc

_N = 10000
_E = 160000
_D = 256
_H = 128
_CHUNK = 128
_TILES = 16
_ROWS_PER_TILE = 80
_IBLK = 16
_NBLK = _ROWS_PER_TILE // _IBLK
_EPAD = _TILES * _ROWS_PER_TILE * _CHUNK
_ACC_ROWS = 10240
_INV_SQRT = 1.0 / math.sqrt(1.0 + 1e-5)


def _seg_aggregate(x_lo, x_hi, src_r, dst_r):
  mesh = plsc.VectorSubcoreMesh(core_axis_name="c", subcore_axis_name="s")

  @functools.partial(
      pl.kernel,
      mesh=mesh,
      out_type=[jax.ShapeDtypeStruct((_ACC_ROWS, _H), jnp.float32)] * 2,
      scratch_types=[
          pltpu.VMEM((_IBLK, _CHUNK), jnp.int32),
          pltpu.VMEM((_IBLK, _CHUNK), jnp.int32),
          pltpu.VMEM((_IBLK, _CHUNK), jnp.int32),
          pltpu.VMEM((_IBLK, _CHUNK), jnp.int32),
          pltpu.VMEM((_CHUNK, _H), jnp.float32),
          pltpu.VMEM((_CHUNK, _H), jnp.float32),
          pltpu.VMEM_SHARED((_ACC_ROWS, _H), jnp.float32),
          pltpu.SemaphoreType.DMA,
          pltpu.SemaphoreType.DMA,
          pltpu.SemaphoreType.DMA,
      ],
  )
  def agg(xlo_hbm, xhi_hbm, src_hbm, dst_hbm, out_lo, out_hi,
          sblk0, dblk0, sblk1, dblk1, rb0, rb1, acc, sem0, sem1, semi):
    c = lax.axis_index("c")
    s = lax.axis_index("s")
    sbufs = (sblk0, sblk1)
    dbufs = (dblk0, dblk1)

    def issue_gather(sb, j, buf, sm):
      @pl.when(c == 0)
      def _():
        pltpu.async_copy(xlo_hbm.at[sb.at[j]], buf, sm)

      @pl.when(c == 1)
      def _():
        pltpu.async_copy(xhi_hbm.at[sb.at[j]], buf, sm)

    def wait_rb(buf, sm):
      pltpu.make_async_copy(xlo_hbm.at[pl.ds(0, _CHUNK)], buf, sm).wait()

    def wait_idx(buf, sm):
      pltpu.make_async_copy(src_hbm.at[pl.ds(0, _IBLK)], buf, sm).wait()

    def issue_idx(b, par):
      base = s * _ROWS_PER_TILE + b * _IBLK
      pltpu.async_copy(src_hbm.at[pl.ds(base, _IBLK)], sbufs[par], semi)
      pltpu.async_copy(dst_hbm.at[pl.ds(base, _IBLK)], dbufs[par], semi)

    def zbody(i, carry):
      for k in range(_H // 16):
        rb0[i, pl.ds(k * 16, 16)] = jnp.zeros((16,), jnp.float32)
      return carry
    lax.fori_loop(0, _CHUNK, zbody, 0)
    issue_idx(0, 0)
    stripe = _ACC_ROWS // _TILES
    for t in range(stripe // _CHUNK):
      pltpu.sync_copy(rb0, acc.at[pl.ds(s * stripe + t * _CHUNK, _CHUNK)])
    wait_idx(sbufs[0], semi)
    wait_idx(dbufs[0], semi)
    plsc.subcore_barrier()

    for b in range(_NBLK):
      sb, db = sbufs[b % 2], dbufs[b % 2]
      if b + 1 < _NBLK:
        issue_idx(b + 1, (b + 1) % 2)
      issue_gather(sb, 0, rb0, sem0)

      def body(k, carry):
        j0 = 2 * k
        issue_gather(sb, j0 + 1, rb1, sem1)
        wait_rb(rb0, sem0)
        pltpu.sync_copy(rb0, acc.at[db.at[j0]], add=True)

        @pl.when(k < _IBLK // 2 - 1)
        def _():
          issue_gather(sb, j0 + 2, rb0, sem0)

        wait_rb(rb1, sem1)
        pltpu.sync_copy(rb1, acc.at[db.at[j0 + 1]], add=True)
        return carry
      lax.fori_loop(0, _IBLK // 2, body, 0)
      if b + 1 < _NBLK:
        wait_idx(sbufs[(b + 1) % 2], semi)
        wait_idx(dbufs[(b + 1) % 2], semi)

    plsc.subcore_barrier()
    rows = _ACC_ROWS // _TILES

    @pl.when(c == 0)
    def _():
      pltpu.sync_copy(acc.at[pl.ds(s * rows, rows)],
                      out_lo.at[pl.ds(s * rows, rows)])

    @pl.when(c == 1)
    def _():
      pltpu.sync_copy(acc.at[pl.ds(s * rows, rows)],
                      out_hi.at[pl.ds(s * rows, rows)])

  return agg(x_lo, x_hi, src_r, dst_r)


_BN_ROWS = 1000


def _mm_bn_relu(a_lo, a_hi, w, g, b):
  def body(lo_ref, hi_ref, w_ref, g_ref, b_ref, olo_ref, ohi_ref):
    a = jnp.concatenate([lo_ref[...], hi_ref[...]], axis=1)
    z = jnp.dot(a, w_ref[...], preferred_element_type=jnp.float32)
    h = jnp.maximum(z * (g_ref[...] * _INV_SQRT) + b_ref[...], 0.0)
    olo_ref[...] = h[:, :_H]
    ohi_ref[...] = h[:, _H:]

  return pl.pallas_call(
      body,
      grid=(_N // _BN_ROWS,),
      in_specs=[
          pl.BlockSpec((_BN_ROWS, _H), lambda i: (i, 0)),
          pl.BlockSpec((_BN_ROWS, _H), lambda i: (i, 0)),
          pl.BlockSpec((_D, _D), lambda i: (0, 0)),
          pl.BlockSpec((1, _D), lambda i: (0, 0)),
          pl.BlockSpec((1, _D), lambda i: (0, 0)),
      ],
      out_specs=[pl.BlockSpec((_BN_ROWS, _H), lambda i: (i, 0))] * 2,
      out_shape=[jax.ShapeDtypeStruct((_N, _H), jnp.float32)] * 2,
  )(a_lo, a_hi, w, g, b)


def _mm_logsoftmax(a_lo, a_hi, w):
  def body(lo_ref, hi_ref, w_ref, o_ref):
    a = jnp.concatenate([lo_ref[...], hi_ref[...]], axis=1)
    z = jnp.dot(a, w_ref[...], preferred_element_type=jnp.float32)
    m = jnp.max(z, axis=1, keepdims=True)
    lse = jnp.log(jnp.sum(jnp.exp(z - m), axis=1, keepdims=True)) + m
    o_ref[...] = z - lse

  return pl.pallas_call(
      body,
      grid=(_N // _BN_ROWS,),
      in_specs=[
          pl.BlockSpec((_BN_ROWS, _H), lambda i: (i, 0)),
          pl.BlockSpec((_BN_ROWS, _H), lambda i: (i, 0)),
          pl.BlockSpec((_D, _D), lambda i: (0, 0)),
      ],
      out_specs=pl.BlockSpec((_BN_ROWS, _D), lambda i: (i, 0)),
      out_shape=jax.ShapeDtypeStruct((_N, _D), jnp.float32),
  )(a_lo, a_hi, w)


def kernel(x, edge_index, W0, W1, W2, gamma0, beta0, gamma1, beta1):
  x = x.astype(jnp.float32)
  src = edge_index[0].astype(jnp.int32)
  dst = edge_index[1].astype(jnp.int32)
  pad = _EPAD - _E
  src_r = jnp.concatenate([src, jnp.zeros((pad,), jnp.int32)]).reshape(-1, _CHUNK)
  dst_r = jnp.concatenate([dst, jnp.full((pad,), _N, jnp.int32)]).reshape(-1, _CHUNK)

  x_lo, x_hi = x[:, :_H], x[:, _H:]
  g0, b0 = gamma0.reshape(1, _D), beta0.reshape(1, _D)
  g1, b1 = gamma1.reshape(1, _D), beta1.reshape(1, _D)

  a_lo, a_hi = _seg_aggregate(x_lo, x_hi, src_r, dst_r)
  h_lo, h_hi = _mm_bn_relu(a_lo, a_hi, W0, g0, b0)
  a_lo, a_hi = _seg_aggregate(h_lo, h_hi, src_r, dst_r)
  h_lo, h_hi = _mm_bn_relu(a_lo, a_hi, W1, g1, b1)
  a_lo, a_hi = _seg_aggregate(h_lo, h_hi, src_r, dst_r)
  return _mm_logsoftmax(a_lo, a_hi, W2)
